# baseline (speedup 1.0000x reference)
import jax
import jax.numpy as jnp
from jax.experimental import pallas as pl
from jax.experimental.pallas import tpu as pltpu

N = 10000
H = 128
BLK = 1024
NB = 10
NPAD = NB * BLK
P1 = NB * NB
GRID = P1 + NB * (NB + 1) // 2
EDGE = N - (NB - 1) * BLK


def _step_ij(s):
    i1, j1 = s // NB, s % NB
    sp = s - P1
    m = jnp.int32(0)
    for k in range(1, NB):
        m = m + (sp >= (k * (k + 1)) // 2).astype(jnp.int32)
    r = (NB - 1) - m
    j2 = r + (sp - (m * (m + 1)) // 2)
    i = jnp.where(s < P1, i1, r)
    j = jnp.where(s < P1, j1, j2)
    return i, j


def _fused_kernel(adj_ref, feats_ref, w1_ref, b1_ref, w2_ref, b2_ref,
                  wo1_ref, wo2_ref, bout_ref, out_ref,
                  y1_s, zbf_s, acc1_s, acc2_s, accout_s, tile_s):
    s = pl.program_id(0)
    i, j = _step_ij(s)
    in_p1 = s < P1

    @pl.when(s == 0)
    def _init():
        y1_s[...] = jnp.zeros((NPAD, H), jnp.bfloat16)
        zbf_s[...] = jnp.zeros((NPAD, H), jnp.bfloat16)
        acc2_s[...] = jnp.zeros((NPAD, H), jnp.float32)
        y1_s[pl.ds(0, N), :] = jnp.dot(feats_ref[...], w1_ref[...],
                                       preferred_element_type=jnp.float32
                                       ).astype(jnp.bfloat16)

    @pl.when(j < NB - 1)
    def _conv():
        tile_s[...] = adj_ref[...].astype(jnp.bfloat16)

    @pl.when(j == NB - 1)
    def _conv_masked():
        col_ids = jax.lax.broadcasted_iota(jnp.int32, (BLK, BLK), 1)
        tile_s[...] = jnp.where(col_ids < EDGE,
                                adj_ref[...].astype(jnp.bfloat16),
                                jnp.bfloat16(0.0))

    tile = tile_s[...]

    @pl.when(in_p1)
    def _pass1():
        c1 = jnp.dot(tile, y1_s[pl.ds(j * BLK, BLK), :],
                     preferred_element_type=jnp.float32)

        @pl.when(j == 0)
        def _():
            acc1_s[...] = c1

        @pl.when(j > 0)
        def _():
            acc1_s[...] = acc1_s[...] + c1

        @pl.when(j < i)
        def _lower_h2():
            acc2_s[pl.ds(i * BLK, BLK), :] = (
                acc2_s[pl.ds(i * BLK, BLK), :]
                + jnp.dot(tile, zbf_s[pl.ds(j * BLK, BLK), :],
                          preferred_element_type=jnp.float32))

        @pl.when(j == NB - 1)
        def _row_end():
            h1 = jnp.maximum(acc1_s[...] + b1_ref[...], 0.0)
            row_ids = jax.lax.broadcasted_iota(jnp.int32, (BLK, H), 0)
            valid = jnp.minimum(N - i * BLK, BLK)
            h1 = jnp.where(row_ids < valid, h1, 0.0)
            zbf_s[pl.ds(i * BLK, BLK), :] = jnp.dot(
                h1, w2_ref[...],
                preferred_element_type=jnp.float32).astype(jnp.bfloat16)
            accout_s[pl.ds(i * BLK, BLK), :] = (
                jnp.dot(h1, wo1_ref[...],
                        preferred_element_type=jnp.float32)
                + bout_ref[...]).astype(jnp.bfloat16)

    @pl.when(~in_p1)
    def _revisit():
        acc2_s[pl.ds(i * BLK, BLK), :] = (
            acc2_s[pl.ds(i * BLK, BLK), :]
            + jnp.dot(tile, zbf_s[pl.ds(j * BLK, BLK), :],
                      preferred_element_type=jnp.float32))

        @pl.when(j == NB - 1)
        def _emit():
            h2 = jnp.maximum(acc2_s[pl.ds(i * BLK, BLK), :] + b2_ref[...],
                             0.0)
            out_ref[...] = (
                jnp.dot(h2, wo2_ref[...], preferred_element_type=jnp.float32)
                + accout_s[pl.ds(i * BLK, BLK), :].astype(jnp.float32))


def _adj_idx(s):
    return _step_ij(s)


def _out_idx(s):
    i, _ = _step_ij(s)
    return (jnp.where(s < P1, NB - 1, i), 0)


@jax.jit
def kernel(feats, adj, W1, b1, W2, b2, Wout, bout):
    full = lambda s: (0, 0)
    small = pl.BlockSpec((H, H), full)
    bias = pl.BlockSpec((1, H), full)

    return pl.pallas_call(
        _fused_kernel,
        grid=(GRID,),
        in_specs=[
            pl.BlockSpec((BLK, BLK), _adj_idx),
            pl.BlockSpec((N, H), full),
            small, bias, small, bias,
            small, small, bias,
        ],
        out_specs=pl.BlockSpec((BLK, H), _out_idx),
        out_shape=jax.ShapeDtypeStruct((N, H), jnp.float32),
        scratch_shapes=[
            pltpu.VMEM((NPAD, H), jnp.bfloat16),
            pltpu.VMEM((NPAD, H), jnp.bfloat16),
            pltpu.VMEM((BLK, H), jnp.float32),
            pltpu.VMEM((NPAD, H), jnp.float32),
            pltpu.VMEM((NPAD, H), jnp.bfloat16),
            pltpu.VMEM((BLK, BLK), jnp.bfloat16),
        ],
        compiler_params=pltpu.CompilerParams(
            dimension_semantics=("arbitrary",),
            vmem_limit_bytes=64 * 1024 * 1024,
        ),
    )(adj, feats, W1, b1.reshape(1, H), W2, b2.reshape(1, H),
      Wout[:H], Wout[H:], bout.reshape(1, H))


# speedup vs baseline: 1.1844x; 1.1844x over previous
import jax
import jax.numpy as jnp
from jax import lax
from jax.experimental import pallas as pl
from jax.experimental.pallas import tpu as pltpu

N = 10000
H = 128
BLK = 200
NB = N // BLK
C = 6
GRID = 2 * NB


def _fused_kernel(adj_ref, feats_ref, w1_ref, b1_ref, w2_ref, b2_ref,
                  wo1_ref, wo2_ref, bout_ref, out_ref,
                  y1_s, z_s, zbf_s, acc_s, cache_s):
    i = pl.program_id(0)

    @pl.when(i == 0)
    def _y1():
        y1_s[...] = jnp.dot(feats_ref[...], w1_ref[...],
                            preferred_element_type=jnp.float32
                            ).astype(jnp.bfloat16)

    @pl.when(i < NB)
    def _pass1():
        h1 = jnp.maximum(
            jnp.dot(adj_ref[...].astype(jnp.bfloat16), y1_s[...],
                    preferred_element_type=jnp.float32) + b1_ref[...], 0.0)
        z_s[pl.ds(i * BLK, BLK), :] = jnp.dot(
            h1, w2_ref[...], preferred_element_type=jnp.float32)
        acc_s[i] = (
            jnp.dot(h1, wo1_ref[...], preferred_element_type=jnp.float32)
            + bout_ref[...]).astype(jnp.bfloat16)

    @pl.when(i < C)
    def _cache():
        cache_s[i] = adj_ref[...].astype(jnp.bfloat16)

    @pl.when(i == NB - 1)
    def _snapshot_zbf():
        zbf_s[...] = z_s[...].astype(jnp.bfloat16)

    def _emit_out(h2, b):
        out_ref[...] = (
            jnp.dot(h2, wo2_ref[...], preferred_element_type=jnp.float32)
            + acc_s[b].astype(jnp.float32))

    @pl.when((i >= NB) & (i < GRID - C))
    def _pass2_streamed():
        b = (GRID - 1) - i
        h2 = jnp.maximum(
            jnp.dot(adj_ref[...].astype(jnp.bfloat16), zbf_s[...],
                    preferred_element_type=jnp.float32) + b2_ref[...], 0.0)
        _emit_out(h2, b)

    @pl.when(i >= GRID - C)
    def _pass2_cached():
        b = i - (GRID - C)
        h2 = jnp.maximum(
            jnp.dot(cache_s[b], zbf_s[...],
                    preferred_element_type=jnp.float32) + b2_ref[...], 0.0)
        _emit_out(h2, b)


def _adj_row(i):
    j = i - NB
    p2 = jnp.where(j < NB - C, NB - 1 - j, C)
    return (jnp.where(i < NB, i, p2), 0)


def _out_row(i):
    j = i - NB
    p2 = jnp.where(j < NB - C, NB - 1 - j, j - (NB - C))
    return (jnp.where(i < NB, NB - 1, p2), 0)


@jax.jit
def kernel(feats, adj, W1, b1, W2, b2, Wout, bout):
    full = lambda i: (0, 0)
    small = pl.BlockSpec((H, H), full)
    bias = pl.BlockSpec((1, H), full)

    return pl.pallas_call(
        _fused_kernel,
        grid=(GRID,),
        in_specs=[
            pl.BlockSpec((BLK, N), _adj_row),
            pl.BlockSpec((N, H), full),
            small, bias, small, bias,
            small, small, bias,
        ],
        out_specs=pl.BlockSpec((BLK, H), _out_row),
        out_shape=jax.ShapeDtypeStruct((N, H), jnp.float32),
        scratch_shapes=[
            pltpu.VMEM((N, H), jnp.bfloat16),
            pltpu.VMEM((N, H), jnp.float32),
            pltpu.VMEM((N, H), jnp.bfloat16),
            pltpu.VMEM((NB, BLK, H), jnp.bfloat16),
            pltpu.VMEM((C, BLK, N), jnp.bfloat16),
        ],
        compiler_params=pltpu.CompilerParams(
            dimension_semantics=("arbitrary",),
            vmem_limit_bytes=64 * 1024 * 1024,
        ),
    )(adj, feats, W1, b1.reshape(1, H), W2, b2.reshape(1, H),
      Wout[:H], Wout[H:], bout.reshape(1, H))


# speedup vs baseline: 1.1893x; 1.0041x over previous
import jax
import jax.numpy as jnp
from jax import lax
from jax.experimental import pallas as pl
from jax.experimental.pallas import tpu as pltpu

N = 10000
H = 128
BLK = 200
NB = N // BLK
C = 8
GRID = 2 * NB


def _fused_kernel(adj_ref, feats_ref, w1_ref, b1_ref, w2_ref, b2_ref,
                  wo1_ref, wo2_ref, bout_ref, out_ref,
                  y1_s, z_s, zbf_s, acc_s, cache_s):
    i = pl.program_id(0)

    @pl.when(i == 0)
    def _y1():
        y1_s[...] = jnp.dot(feats_ref[...], w1_ref[...],
                            preferred_element_type=jnp.float32
                            ).astype(jnp.bfloat16)

    @pl.when(i < NB)
    def _pass1():
        h1 = jnp.maximum(
            jnp.dot(adj_ref[...].astype(jnp.bfloat16), y1_s[...],
                    preferred_element_type=jnp.float32) + b1_ref[...], 0.0)
        z_s[pl.ds(i * BLK, BLK), :] = jnp.dot(
            h1, w2_ref[...], preferred_element_type=jnp.float32)
        acc_s[i] = (
            jnp.dot(h1, wo1_ref[...], preferred_element_type=jnp.float32)
            + bout_ref[...]).astype(jnp.bfloat16)

    @pl.when(i < C)
    def _cache():
        cache_s[i] = adj_ref[...].astype(jnp.bfloat16)

    @pl.when(i == NB - 1)
    def _snapshot_zbf():
        zbf_s[...] = z_s[...].astype(jnp.bfloat16)

    def _emit_out(h2, b):
        out_ref[...] = (
            jnp.dot(h2, wo2_ref[...], preferred_element_type=jnp.float32)
            + acc_s[b].astype(jnp.float32))

    @pl.when((i >= NB) & (i < GRID - C))
    def _pass2_streamed():
        b = (GRID - 1) - i
        h2 = jnp.maximum(
            jnp.dot(adj_ref[...].astype(jnp.bfloat16), zbf_s[...],
                    preferred_element_type=jnp.float32) + b2_ref[...], 0.0)
        _emit_out(h2, b)

    @pl.when(i >= GRID - C)
    def _pass2_cached():
        b = i - (GRID - C)
        h2 = jnp.maximum(
            jnp.dot(cache_s[b], zbf_s[...],
                    preferred_element_type=jnp.float32) + b2_ref[...], 0.0)
        _emit_out(h2, b)


def _adj_row(i):
    j = i - NB
    p2 = jnp.where(j < NB - C, NB - 1 - j, C)
    return (jnp.where(i < NB, i, p2), 0)


def _out_row(i):
    j = i - NB
    p2 = jnp.where(j < NB - C, NB - 1 - j, j - (NB - C))
    return (jnp.where(i < NB, NB - 1, p2), 0)


@jax.jit
def kernel(feats, adj, W1, b1, W2, b2, Wout, bout):
    full = lambda i: (0, 0)
    small = pl.BlockSpec((H, H), full)
    bias = pl.BlockSpec((1, H), full)

    return pl.pallas_call(
        _fused_kernel,
        grid=(GRID,),
        in_specs=[
            pl.BlockSpec((BLK, N), _adj_row),
            pl.BlockSpec((N, H), full),
            small, bias, small, bias,
            small, small, bias,
        ],
        out_specs=pl.BlockSpec((BLK, H), _out_row),
        out_shape=jax.ShapeDtypeStruct((N, H), jnp.float32),
        scratch_shapes=[
            pltpu.VMEM((N, H), jnp.bfloat16),
            pltpu.VMEM((N, H), jnp.float32),
            pltpu.VMEM((N, H), jnp.bfloat16),
            pltpu.VMEM((NB, BLK, H), jnp.bfloat16),
            pltpu.VMEM((C, BLK, N), jnp.bfloat16),
        ],
        compiler_params=pltpu.CompilerParams(
            dimension_semantics=("arbitrary",),
            vmem_limit_bytes=100 * 1024 * 1024,
        ),
    )(adj, feats, W1, b1.reshape(1, H), W2, b2.reshape(1, H),
      Wout[:H], Wout[H:], bout.reshape(1, H))
